# baseline (speedup 1.0000x reference)
import jax
import jax.numpy as jnp
from jax import lax
from jax.experimental import pallas as pl
from jax.experimental.pallas import tpu as pltpu
from jax.experimental.pallas import tpu_sc as plsc

N = 50000
E = 800000
H = 64
NG = 16
CFG = 24
NOPS = 120
D0 = 160

CH = 128
SUP = 16
NSUP = 25
TILES = 16
EPT = CH * SUP * NSUP
E_PAD = EPT * TILES
ROWS_PAD = E_PAD // CH
ACC_ROWS = 51200
TPT = ACC_ROWS // TILES
DUMP = N
ZR = 64

BT = 1000
NB_TC = N // BT

_INTERPRET = False


def _make_agg_body(width, cps, slots):
    def body(y_hbm, g_hbm, s_hbm, z_hbm, out_hbm, acc, gbuf, sbuf, rows, zbuf,
             gsem, ssem):
        c = lax.axis_index("c")
        s = lax.axis_index("s")
        pltpu.sync_copy(z_hbm, zbuf)
        row_t = s * (SUP * NSUP)

        for j in range(cps):
            chunk = c * cps + j
            for i in range(TPT // ZR):
                pltpu.sync_copy(zbuf, acc.at[pl.ds(s * TPT + i * ZR, ZR)])
            plsc.subcore_barrier()
            gbase = chunk * ROWS_PAD

            def sup_body(sup, carry):
                r0 = row_t + sup * SUP
                pltpu.sync_copy(g_hbm.at[pl.ds(gbase + r0, SUP)], gbuf)
                pltpu.sync_copy(s_hbm.at[pl.ds(r0, SUP)], sbuf)
                gh = [None] * slots
                sh = [None] * slots

                def scat(w):
                    wsl = w % slots
                    gh[wsl].wait()
                    sh[wsl] = pltpu.async_copy(
                        rows.at[wsl], acc.at[sbuf.at[w]], ssem.at[wsl],
                        add=True)

                for k in range(SUP):
                    sl = k % slots
                    if k >= slots:
                        sh[sl].wait()
                    gh[sl] = pltpu.async_copy(
                        y_hbm.at[gbuf.at[k]], rows.at[sl], gsem.at[sl])
                    if k >= slots - 1:
                        scat(k - (slots - 1))
                for w in range(max(SUP - slots + 1, 0), SUP):
                    scat(w)
                for t in range(min(slots, SUP)):
                    if sh[t] is not None:
                        sh[t].wait()
                return carry

            lax.fori_loop(0, NSUP, sup_body, 0)
            plsc.subcore_barrier()
            pltpu.sync_copy(
                acc.at[pl.ds(s * TPT, TPT)],
                out_hbm.at[pl.ds(chunk * ACC_ROWS + s * TPT, TPT)])
            plsc.subcore_barrier()

    return body


def _cnt_body(c_hbm, z_hbm, o_hbm, out_hbm, acc, cbuf, ones, zbuf, csem):
    c = lax.axis_index("c")
    s = lax.axis_index("s")
    pltpu.sync_copy(z_hbm, zbuf)
    for i in range(TPT // ZR):
        pltpu.sync_copy(zbuf, acc.at[pl.ds(s * TPT + i * ZR, ZR)])
    pltpu.sync_copy(o_hbm, ones)
    plsc.subcore_barrier()

    row_t = s * (SUP * NSUP)

    def sup_body(sup, carry):
        r0 = row_t + sup * SUP
        pltpu.sync_copy(c_hbm.at[pl.ds(c * ROWS_PAD + r0, SUP)], cbuf)
        handles = []
        for k in range(SUP):
            handles.append(
                pltpu.async_copy(ones, acc.at[cbuf.at[k]], csem, add=True))
        for h in handles:
            h.wait()
        return carry

    lax.fori_loop(0, NSUP, sup_body, 0)
    plsc.subcore_barrier()
    pltpu.sync_copy(acc.at[pl.ds(s * TPT, TPT)],
                    out_hbm.at[pl.ds(c * ACC_ROWS + s * TPT, TPT)])


def _sc_mesh():
    return plsc.VectorSubcoreMesh(core_axis_name="c", subcore_axis_name="s")


def _agg(y_flat, garr, sarr, zeros, width, cps, slots):
    return pl.kernel(
        _make_agg_body(width, cps, slots),
        out_type=jax.ShapeDtypeStruct((2 * cps * ACC_ROWS, width),
                                      jnp.float32),
        mesh=_sc_mesh(),
        scratch_types=[
            pltpu.VMEM_SHARED((ACC_ROWS, width), jnp.float32),
            pltpu.VMEM((SUP, CH), jnp.int32),
            pltpu.VMEM((SUP, CH), jnp.int32),
            pltpu.VMEM((slots, CH, width), jnp.float32),
            pltpu.VMEM((ZR, width), jnp.float32),
            pltpu.SemaphoreType.DMA((slots,)),
            pltpu.SemaphoreType.DMA((slots,)),
        ],
        compiler_params=pltpu.CompilerParams(use_tc_tiling_on_sc=False),
        interpret=_INTERPRET,
    )(y_flat, garr, sarr, zeros)


def _cnt(carr, zeros16, ones16):
    return pl.kernel(
        _cnt_body,
        out_type=jax.ShapeDtypeStruct((2 * ACC_ROWS, 16), jnp.float32),
        mesh=_sc_mesh(),
        scratch_types=[
            pltpu.VMEM_SHARED((ACC_ROWS, 16), jnp.float32),
            pltpu.VMEM((SUP, CH), jnp.int32),
            pltpu.VMEM((CH, 16), jnp.float32),
            pltpu.VMEM((ZR, 16), jnp.float32),
            pltpu.SemaphoreType.DMA,
        ],
        compiler_params=pltpu.CompilerParams(use_tc_tiling_on_sc=False),
        interpret=_INTERPRET,
    )(carr, zeros16, ones16)


def _dot(a, b):
    return jnp.dot(a, b, preferred_element_type=jnp.float32)


def _dot_hi(a, b):
    return jnp.dot(a, b, preferred_element_type=jnp.float32,
                   precision=lax.Precision.HIGHEST)


def _emb_body(nf_ref, op_ref, sh_ref, ope_ref, she_ref, xs_ref):
    nf = nf_ref[...]
    opv = op_ref[0, 0, :]
    shv = sh_ref[0, 0, :]
    ohop = (opv[:, None] == lax.broadcasted_iota(jnp.int32, (BT, 128), 1)
            ).astype(jnp.float32)
    ohsh = (shv[:, None] == lax.broadcasted_iota(jnp.int32, (BT, 8), 1)
            ).astype(jnp.float32)
    emb = jnp.concatenate(
        [_dot_hi(ohop, ope_ref[...]), _dot_hi(ohsh, she_ref[...])], axis=1)
    for j in range(8):
        xs_ref[j:j + 1] = nf[:, 16 * j:16 * (j + 1)][None]
    xs_ref[8:9] = jnp.concatenate([nf[:, 128:139], emb[:, :5]], axis=1)[None]
    xs_ref[9:10] = jnp.concatenate(
        [emb[:, 5:], jnp.zeros((BT, 9), jnp.float32)], axis=1)[None]


def _cat_mean(agg_ref, cnt_ref, col, nchunk):
    a = agg_ref[...]
    cat = jnp.concatenate([a[j] for j in range(nchunk)], axis=1)
    return cat / jnp.maximum(cnt_ref[0, :, col:col + 1], 1.0)


def _mk_layer_body(nchunk, din, out_split):
    def body(af_ref, ab_ref, xs_ref, cnf_ref, cnb_ref, wnf_ref, wrf_ref,
             bf_ref, wnb_ref, wrb_ref, bb_ref, o_ref):
        meanf = _cat_mean(af_ref, cnf_ref, 0, nchunk)
        meanb = _cat_mean(ab_ref, cnb_ref, 0, nchunk)
        xs = xs_ref[...]
        x = jnp.concatenate([xs[j] for j in range(nchunk)], axis=1)
        meanf = meanf[:, :din]
        meanb = meanb[:, :din]
        x = x[:, :din]
        hf = _dot(meanf, wnf_ref[...]) + _dot(x, wrf_ref[...]) + bf_ref[...]
        hb = _dot(meanb, wnb_ref[...]) + _dot(x, wrb_ref[...]) + bb_ref[...]
        x = jnp.maximum(hf + hb, 0.0)
        if out_split:
            o_ref[0:1] = x[:, :32][None]
            o_ref[1:2] = x[:, 32:][None]
        else:
            o_ref[...] = x
    return body


def _pool_body(x_ref, b3_ref, cfg_ref, w1_ref, b1_ref, w2_ref, b2_ref,
               out_ref, sums, maxs, cnts):
    i = pl.program_id(0)

    @pl.when(i == 0)
    def _init():
        sums[...] = jnp.zeros((NG, H), jnp.float32)
        maxs[...] = jnp.full((NG, H), -3e38, jnp.float32)
        cnts[...] = jnp.zeros((NG, H), jnp.float32)

    x = x_ref[...]
    bv = b3_ref[0, 0, :]
    oh = (bv[:, None] == lax.broadcasted_iota(jnp.int32, (BT, NG), 1)
          ).astype(jnp.float32)
    sums[...] += lax.dot_general(oh, x, (((0,), (0,)), ((), ())),
                                 preferred_element_type=jnp.float32,
                                 precision=lax.Precision.HIGHEST)
    cnts[...] += jnp.broadcast_to(jnp.sum(oh, axis=0)[:, None], (NG, H))
    mx = maxs[...]
    upd = []
    for g in range(NG):
        mg = jnp.max(jnp.where(oh[:, g:g + 1] > 0.5, x, -3e38), axis=0)
        upd.append(jnp.maximum(mx[g], mg))
    maxs[...] = jnp.stack(upd, axis=0)

    @pl.when(i == NB_TC - 1)
    def _final():
        avg = sums[...] / jnp.maximum(cnts[...], 1.0)
        cat = jnp.concatenate([avg, maxs[...], cfg_ref[...]], axis=1)
        h = jnp.maximum(_dot(cat, w1_ref[...]) + b1_ref[...], 0.0)
        out_ref[...] = _dot(h, w2_ref[...]) + b2_ref[...]


def _full(shape):
    return pl.BlockSpec(shape, lambda i: (0,) * len(shape))


_NODE140 = pl.BlockSpec((BT, 140), lambda i: (i, 0))
_IDX3 = pl.BlockSpec((1, 1, BT), lambda i: (i, 0, 0))
_NODE64 = pl.BlockSpec((BT, H), lambda i: (i, 0))
_CIN = pl.BlockSpec((1, BT, 16), lambda i: (0, i, 0))
_COUT = pl.BlockSpec((1, BT, 16), lambda i: (1, i, 0))


def _chunk3(nc, w):
    return pl.BlockSpec((nc, BT, w), lambda i: (0, i, 0))


def _emb(nf, op3, sh3, ope, she):
    return pl.pallas_call(
        _emb_body,
        grid=(NB_TC,),
        in_specs=[_NODE140, _IDX3, _IDX3, _full((128, 8)), _full((8, 4))],
        out_specs=[_chunk3(10, 16)],
        out_shape=[jax.ShapeDtypeStruct((10, N, 16), jnp.float32)],
        interpret=_INTERPRET,
    )(nf, op3, sh3, ope, she)[0]


def _layer(aggf, aggb, xs, cnt, wnf, wrf, bf, wnb, wrb, bb, nchunk, width,
           out_split):
    din = wrf.shape[0]
    if out_split:
        out_spec = _chunk3(2, 32)
        out_shape = jax.ShapeDtypeStruct((2, N, 32), jnp.float32)
    else:
        out_spec = _NODE64
        out_shape = jax.ShapeDtypeStruct((N, H), jnp.float32)
    return pl.pallas_call(
        _mk_layer_body(nchunk, din, out_split),
        grid=(NB_TC,),
        in_specs=[_chunk3(nchunk, width), _chunk3(nchunk, width),
                  _chunk3(nchunk, width), _CIN, _COUT,
                  _full((din, H)), _full((din, H)), _full((1, H)),
                  _full((din, H)), _full((din, H)), _full((1, H))],
        out_specs=[out_spec],
        out_shape=[out_shape],
        interpret=_INTERPRET,
    )(aggf, aggb, xs, cnt, cnt, wnf, wrf, bf, wnb, wrb, bb)[0]


def _pool(x3, b3, cfg, w1, b1, w2, b2):
    return pl.pallas_call(
        _pool_body,
        grid=(NB_TC,),
        in_specs=[_NODE64, _IDX3, _full((NG, CFG)), _full((2 * H + CFG, H)),
                  _full((1, H)), _full((H, 1)), _full((1, 1))],
        out_specs=pl.BlockSpec((NG, 1), lambda i: (0, 0)),
        out_shape=jax.ShapeDtypeStruct((NG, 1), jnp.float32),
        scratch_shapes=[pltpu.VMEM((NG, H), jnp.float32)] * 3,
        interpret=_INTERPRET,
    )(x3, b3, cfg, w1, b1, w2, b2)


def kernel(node_feat, node_opcode, edge_index, config_feat, n_configs, batch,
           params):
    f32 = jnp.float32
    i32 = jnp.int32
    shape_idx = node_feat[:, -1].astype(i32)
    op3 = node_opcode.reshape(NB_TC, 1, BT)
    sh3 = shape_idx.reshape(NB_TC, 1, BT)
    b3 = batch.reshape(NB_TC, 1, BT)

    L = params["layers"]
    ope = jnp.zeros((128, 8), f32).at[:NOPS].set(params["op_emb"])

    def pad0(w):
        return jnp.concatenate([w, jnp.zeros((D0 - 151, H), f32)], axis=0)

    src = edge_index[:, 0]
    dst = edge_index[:, 1]
    pad_g = jnp.zeros((E_PAD - E,), i32)
    pad_s = jnp.full((E_PAD - E,), DUMP, i32)
    srcg = jnp.concatenate([src, pad_g])
    dstg = jnp.concatenate([dst, pad_g])
    srcs = jnp.concatenate([src, pad_s])
    dsts = jnp.concatenate([dst, pad_s])
    off10 = (jnp.arange(10, dtype=i32) * N)[:, None]
    off2 = (jnp.arange(2, dtype=i32) * N)[:, None]
    garrF10 = (srcg[None] + off10).reshape(10 * ROWS_PAD, CH)
    garrB10 = (dstg[None] + off10).reshape(10 * ROWS_PAD, CH)
    garrF2 = (srcg[None] + off2).reshape(2 * ROWS_PAD, CH)
    garrB2 = (dstg[None] + off2).reshape(2 * ROWS_PAD, CH)
    sarrF = dsts.reshape(ROWS_PAD, CH)
    sarrB = srcs.reshape(ROWS_PAD, CH)
    carr = jnp.concatenate([dsts, srcs]).reshape(2 * ROWS_PAD, CH)
    zeros32 = jnp.zeros((ZR, 32), f32)
    zeros16 = jnp.zeros((ZR, 16), f32)
    ones16 = jnp.ones((CH, 16), f32)

    cnt = _cnt(carr, zeros16, ones16).reshape(2, ACC_ROWS, 16)

    xs = _emb(node_feat, op3, sh3, ope, params["shape_emb"])

    for l in range(3):
        lay = L[l]
        if l == 0:
            y = xs.reshape(10 * N, 16)
            aggf = _agg(y, garrF10, sarrF, zeros16, 16, 5, 8)
            aggb = _agg(y, garrB10, sarrB, zeros16, 16, 5, 8)
            aggf = aggf.reshape(10, ACC_ROWS, 16)
            aggb = aggb.reshape(10, ACC_ROWS, 16)
            wnf, wrf = pad0(lay["Wn_f"]), pad0(lay["Wr_f"])
            wnb, wrb = pad0(lay["Wn_b"]), pad0(lay["Wr_b"])
            nchunk, width = 10, 16
        else:
            y = xs.reshape(2 * N, 32)
            aggf = _agg(y, garrF2, sarrF, zeros32, 32, 1, 5)
            aggb = _agg(y, garrB2, sarrB, zeros32, 32, 1, 5)
            aggf = aggf.reshape(2, ACC_ROWS, 32)
            aggb = aggb.reshape(2, ACC_ROWS, 32)
            wnf, wrf = lay["Wn_f"], lay["Wr_f"]
            wnb, wrb = lay["Wn_b"], lay["Wr_b"]
            nchunk, width = 2, 32
        xs = _layer(aggf, aggb, xs, cnt, wnf, wrf, lay["b_f"][None],
                    wnb, wrb, lay["b_b"][None], nchunk, width,
                    out_split=(l < 2))

    out = _pool(xs, b3, config_feat, params["W1"], params["b1"][None],
                params["W2"], params["b2"][None])
    return out[:, 0]


# speedup vs baseline: 1.0040x; 1.0040x over previous
import jax
import jax.numpy as jnp
from jax import lax
from jax.experimental import pallas as pl
from jax.experimental.pallas import tpu as pltpu
from jax.experimental.pallas import tpu_sc as plsc

N = 50000
E = 800000
H = 64
NG = 16
CFG = 24
NOPS = 120
D0 = 160

CH = 128
SUP = 16
NSUP = 25
TILES = 16
EPT = CH * SUP * NSUP
E_PAD = EPT * TILES
ROWS_PAD = E_PAD // CH
ACC_ROWS = 51200
TPT = ACC_ROWS // TILES
DUMP = N
ZR = 64

BT = 1000
NB_TC = N // BT

_INTERPRET = False


def _make_agg_body(width, cps, slots):
    def body(y_hbm, g_hbm, s_hbm, z_hbm, out_hbm, acc, gbuf, sbuf, rows, zbuf,
             gsem, ssem):
        c = lax.axis_index("c")
        s = lax.axis_index("s")
        pltpu.sync_copy(z_hbm, zbuf)
        row_t = s * (SUP * NSUP)

        for j in range(cps):
            chunk = c * cps + j
            for i in range(TPT // ZR):
                pltpu.sync_copy(zbuf, acc.at[pl.ds(s * TPT + i * ZR, ZR)])
            plsc.subcore_barrier()
            gbase = chunk * ROWS_PAD

            def sup_body(sup, carry):
                r0 = row_t + sup * SUP
                pltpu.sync_copy(g_hbm.at[pl.ds(gbase + r0, SUP)], gbuf)
                pltpu.sync_copy(s_hbm.at[pl.ds(r0, SUP)], sbuf)
                gh = [None] * slots
                sh = [None] * slots

                def scat(w):
                    wsl = w % slots
                    gh[wsl].wait()
                    sh[wsl] = pltpu.async_copy(
                        rows.at[wsl], acc.at[sbuf.at[w]], ssem.at[wsl],
                        add=True)

                for k in range(SUP):
                    sl = k % slots
                    if k >= slots:
                        sh[sl].wait()
                    gh[sl] = pltpu.async_copy(
                        y_hbm.at[gbuf.at[k]], rows.at[sl], gsem.at[sl])
                    if k >= slots - 1:
                        scat(k - (slots - 1))
                for w in range(max(SUP - slots + 1, 0), SUP):
                    scat(w)
                for t in range(min(slots, SUP)):
                    if sh[t] is not None:
                        sh[t].wait()
                return carry

            lax.fori_loop(0, NSUP, sup_body, 0)
            plsc.subcore_barrier()
            pltpu.sync_copy(
                acc.at[pl.ds(s * TPT, TPT)],
                out_hbm.at[pl.ds(chunk * ACC_ROWS + s * TPT, TPT)])
            plsc.subcore_barrier()

    return body


def _cnt_body(c_hbm, z_hbm, o_hbm, out_hbm, acc, cbuf, ones, zbuf, csem):
    c = lax.axis_index("c")
    s = lax.axis_index("s")
    pltpu.sync_copy(z_hbm, zbuf)
    for i in range(TPT // ZR):
        pltpu.sync_copy(zbuf, acc.at[pl.ds(s * TPT + i * ZR, ZR)])
    pltpu.sync_copy(o_hbm, ones)
    plsc.subcore_barrier()

    row_t = s * (SUP * NSUP)

    def sup_body(sup, carry):
        r0 = row_t + sup * SUP
        pltpu.sync_copy(c_hbm.at[pl.ds(c * ROWS_PAD + r0, SUP)], cbuf)
        handles = []
        for k in range(SUP):
            handles.append(
                pltpu.async_copy(ones, acc.at[cbuf.at[k]], csem, add=True))
        for h in handles:
            h.wait()
        return carry

    lax.fori_loop(0, NSUP, sup_body, 0)
    plsc.subcore_barrier()
    pltpu.sync_copy(acc.at[pl.ds(s * TPT, TPT)],
                    out_hbm.at[pl.ds(c * ACC_ROWS + s * TPT, TPT)])


def _sc_mesh():
    return plsc.VectorSubcoreMesh(core_axis_name="c", subcore_axis_name="s")


def _agg(y_flat, garr, sarr, zeros, width, cps, slots):
    return pl.kernel(
        _make_agg_body(width, cps, slots),
        out_type=jax.ShapeDtypeStruct((2 * cps * ACC_ROWS, width),
                                      jnp.float32),
        mesh=_sc_mesh(),
        scratch_types=[
            pltpu.VMEM_SHARED((ACC_ROWS, width), jnp.float32),
            pltpu.VMEM((SUP, CH), jnp.int32),
            pltpu.VMEM((SUP, CH), jnp.int32),
            pltpu.VMEM((slots, CH, width), jnp.float32),
            pltpu.VMEM((ZR, width), jnp.float32),
            pltpu.SemaphoreType.DMA((slots,)),
            pltpu.SemaphoreType.DMA((slots,)),
        ],
        compiler_params=pltpu.CompilerParams(use_tc_tiling_on_sc=False),
        interpret=_INTERPRET,
    )(y_flat, garr, sarr, zeros)


def _cnt(carr, zeros16, ones16):
    return pl.kernel(
        _cnt_body,
        out_type=jax.ShapeDtypeStruct((2 * ACC_ROWS, 16), jnp.float32),
        mesh=_sc_mesh(),
        scratch_types=[
            pltpu.VMEM_SHARED((ACC_ROWS, 16), jnp.float32),
            pltpu.VMEM((SUP, CH), jnp.int32),
            pltpu.VMEM((CH, 16), jnp.float32),
            pltpu.VMEM((ZR, 16), jnp.float32),
            pltpu.SemaphoreType.DMA,
        ],
        compiler_params=pltpu.CompilerParams(use_tc_tiling_on_sc=False),
        interpret=_INTERPRET,
    )(carr, zeros16, ones16)


def _dot(a, b):
    return jnp.dot(a, b, preferred_element_type=jnp.float32)


def _dot_hi(a, b):
    return jnp.dot(a, b, preferred_element_type=jnp.float32,
                   precision=lax.Precision.HIGHEST)


def _emb_body(nf_ref, op_ref, sh_ref, ope_ref, she_ref, xs_ref):
    nf = nf_ref[...]
    opv = op_ref[0, 0, :]
    shv = sh_ref[0, 0, :]
    ohop = (opv[:, None] == lax.broadcasted_iota(jnp.int32, (BT, 128), 1)
            ).astype(jnp.float32)
    ohsh = (shv[:, None] == lax.broadcasted_iota(jnp.int32, (BT, 8), 1)
            ).astype(jnp.float32)
    emb = jnp.concatenate(
        [_dot_hi(ohop, ope_ref[...]), _dot_hi(ohsh, she_ref[...])], axis=1)
    for j in range(8):
        xs_ref[j:j + 1] = nf[:, 16 * j:16 * (j + 1)][None]
    xs_ref[8:9] = jnp.concatenate([nf[:, 128:139], emb[:, :5]], axis=1)[None]
    xs_ref[9:10] = jnp.concatenate(
        [emb[:, 5:], jnp.zeros((BT, 9), jnp.float32)], axis=1)[None]


def _cat_mean(agg_ref, cnt_ref, col, nchunk):
    a = agg_ref[...]
    cat = jnp.concatenate([a[j] for j in range(nchunk)], axis=1)
    return cat / jnp.maximum(cnt_ref[0, :, col:col + 1], 1.0)


def _mk_layer_body(nchunk, din, out_split):
    def body(af_ref, ab_ref, xs_ref, cnf_ref, cnb_ref, wnf_ref, wrf_ref,
             bf_ref, wnb_ref, wrb_ref, bb_ref, o_ref):
        meanf = _cat_mean(af_ref, cnf_ref, 0, nchunk)
        meanb = _cat_mean(ab_ref, cnb_ref, 0, nchunk)
        xs = xs_ref[...]
        x = jnp.concatenate([xs[j] for j in range(nchunk)], axis=1)
        meanf = meanf[:, :din]
        meanb = meanb[:, :din]
        x = x[:, :din]
        hf = _dot(meanf, wnf_ref[...]) + _dot(x, wrf_ref[...]) + bf_ref[...]
        hb = _dot(meanb, wnb_ref[...]) + _dot(x, wrb_ref[...]) + bb_ref[...]
        x = jnp.maximum(hf + hb, 0.0)
        if out_split:
            o_ref[0:1] = x[:, :32][None]
            o_ref[1:2] = x[:, 32:][None]
        else:
            o_ref[...] = x
    return body


def _pool_body(x_ref, b3_ref, cfg_ref, w1_ref, b1_ref, w2_ref, b2_ref,
               out_ref, sums, maxs, cnts):
    i = pl.program_id(0)

    @pl.when(i == 0)
    def _init():
        sums[...] = jnp.zeros((NG, H), jnp.float32)
        maxs[...] = jnp.full((NG, H), -3e38, jnp.float32)
        cnts[...] = jnp.zeros((NG, H), jnp.float32)

    x = x_ref[...]
    bv = b3_ref[0, 0, :]
    oh = (bv[:, None] == lax.broadcasted_iota(jnp.int32, (BT, NG), 1)
          ).astype(jnp.float32)
    sums[...] += lax.dot_general(oh, x, (((0,), (0,)), ((), ())),
                                 preferred_element_type=jnp.float32,
                                 precision=lax.Precision.HIGHEST)
    cnts[...] += jnp.broadcast_to(jnp.sum(oh, axis=0)[:, None], (NG, H))
    mx = maxs[...]
    upd = []
    for g in range(NG):
        mg = jnp.max(jnp.where(oh[:, g:g + 1] > 0.5, x, -3e38), axis=0)
        upd.append(jnp.maximum(mx[g], mg))
    maxs[...] = jnp.stack(upd, axis=0)

    @pl.when(i == NB_TC - 1)
    def _final():
        avg = sums[...] / jnp.maximum(cnts[...], 1.0)
        cat = jnp.concatenate([avg, maxs[...], cfg_ref[...]], axis=1)
        h = jnp.maximum(_dot(cat, w1_ref[...]) + b1_ref[...], 0.0)
        out_ref[...] = _dot(h, w2_ref[...]) + b2_ref[...]


def _full(shape):
    return pl.BlockSpec(shape, lambda i: (0,) * len(shape))


_NODE140 = pl.BlockSpec((BT, 140), lambda i: (i, 0))
_IDX3 = pl.BlockSpec((1, 1, BT), lambda i: (i, 0, 0))
_NODE64 = pl.BlockSpec((BT, H), lambda i: (i, 0))
_CIN = pl.BlockSpec((1, BT, 16), lambda i: (0, i, 0))
_COUT = pl.BlockSpec((1, BT, 16), lambda i: (1, i, 0))


def _chunk3(nc, w):
    return pl.BlockSpec((nc, BT, w), lambda i: (0, i, 0))


def _emb(nf, op3, sh3, ope, she):
    return pl.pallas_call(
        _emb_body,
        grid=(NB_TC,),
        in_specs=[_NODE140, _IDX3, _IDX3, _full((128, 8)), _full((8, 4))],
        out_specs=[_chunk3(10, 16)],
        out_shape=[jax.ShapeDtypeStruct((10, N, 16), jnp.float32)],
        interpret=_INTERPRET,
    )(nf, op3, sh3, ope, she)[0]


def _layer(aggf, aggb, xs, cnt, wnf, wrf, bf, wnb, wrb, bb, nchunk, width,
           out_split):
    din = wrf.shape[0]
    if out_split:
        out_spec = _chunk3(2, 32)
        out_shape = jax.ShapeDtypeStruct((2, N, 32), jnp.float32)
    else:
        out_spec = _NODE64
        out_shape = jax.ShapeDtypeStruct((N, H), jnp.float32)
    return pl.pallas_call(
        _mk_layer_body(nchunk, din, out_split),
        grid=(NB_TC,),
        in_specs=[_chunk3(nchunk, width), _chunk3(nchunk, width),
                  _chunk3(nchunk, width), _CIN, _COUT,
                  _full((din, H)), _full((din, H)), _full((1, H)),
                  _full((din, H)), _full((din, H)), _full((1, H))],
        out_specs=[out_spec],
        out_shape=[out_shape],
        interpret=_INTERPRET,
    )(aggf, aggb, xs, cnt, cnt, wnf, wrf, bf, wnb, wrb, bb)[0]


def _pool(x3, b3, cfg, w1, b1, w2, b2):
    return pl.pallas_call(
        _pool_body,
        grid=(NB_TC,),
        in_specs=[_NODE64, _IDX3, _full((NG, CFG)), _full((2 * H + CFG, H)),
                  _full((1, H)), _full((H, 1)), _full((1, 1))],
        out_specs=pl.BlockSpec((NG, 1), lambda i: (0, 0)),
        out_shape=jax.ShapeDtypeStruct((NG, 1), jnp.float32),
        scratch_shapes=[pltpu.VMEM((NG, H), jnp.float32)] * 3,
        interpret=_INTERPRET,
    )(x3, b3, cfg, w1, b1, w2, b2)


def kernel(node_feat, node_opcode, edge_index, config_feat, n_configs, batch,
           params):
    f32 = jnp.float32
    i32 = jnp.int32
    shape_idx = node_feat[:, -1].astype(i32)
    op3 = node_opcode.reshape(NB_TC, 1, BT)
    sh3 = shape_idx.reshape(NB_TC, 1, BT)
    b3 = batch.reshape(NB_TC, 1, BT)

    L = params["layers"]
    ope = jnp.zeros((128, 8), f32).at[:NOPS].set(params["op_emb"])

    def pad0(w):
        return jnp.concatenate([w, jnp.zeros((D0 - 151, H), f32)], axis=0)

    src = edge_index[:, 0]
    dst = edge_index[:, 1]
    pad_g = jnp.zeros((E_PAD - E,), i32)
    pad_s = jnp.full((E_PAD - E,), DUMP, i32)
    srcg = jnp.concatenate([src, pad_g])
    dstg = jnp.concatenate([dst, pad_g])
    srcs = jnp.concatenate([src, pad_s])
    dsts = jnp.concatenate([dst, pad_s])
    off10 = (jnp.arange(10, dtype=i32) * N)[:, None]
    off2 = (jnp.arange(2, dtype=i32) * N)[:, None]
    garrF10 = (srcg[None] + off10).reshape(10 * ROWS_PAD, CH)
    garrB10 = (dstg[None] + off10).reshape(10 * ROWS_PAD, CH)
    garrF2 = (srcg[None] + off2).reshape(2 * ROWS_PAD, CH)
    garrB2 = (dstg[None] + off2).reshape(2 * ROWS_PAD, CH)
    sarrF = dsts.reshape(ROWS_PAD, CH)
    sarrB = srcs.reshape(ROWS_PAD, CH)
    carr = jnp.concatenate([dsts, srcs]).reshape(2 * ROWS_PAD, CH)
    zeros32 = jnp.zeros((ZR, 32), f32)
    zeros16 = jnp.zeros((ZR, 16), f32)
    ones16 = jnp.ones((CH, 16), f32)

    cnt = _cnt(carr, zeros16, ones16).reshape(2, ACC_ROWS, 16)

    xs = _emb(node_feat, op3, sh3, ope, params["shape_emb"])

    for l in range(3):
        lay = L[l]
        if l == 0:
            y = xs.reshape(10 * N, 16)
            aggf = _agg(y, garrF10, sarrF, zeros16, 16, 5, 12)
            aggb = _agg(y, garrB10, sarrB, zeros16, 16, 5, 12)
            aggf = aggf.reshape(10, ACC_ROWS, 16)
            aggb = aggb.reshape(10, ACC_ROWS, 16)
            wnf, wrf = pad0(lay["Wn_f"]), pad0(lay["Wr_f"])
            wnb, wrb = pad0(lay["Wn_b"]), pad0(lay["Wr_b"])
            nchunk, width = 10, 16
        else:
            y = xs.reshape(2 * N, 32)
            aggf = _agg(y, garrF2, sarrF, zeros32, 32, 1, 5)
            aggb = _agg(y, garrB2, sarrB, zeros32, 32, 1, 5)
            aggf = aggf.reshape(2, ACC_ROWS, 32)
            aggb = aggb.reshape(2, ACC_ROWS, 32)
            wnf, wrf = lay["Wn_f"], lay["Wr_f"]
            wnb, wrb = lay["Wn_b"], lay["Wr_b"]
            nchunk, width = 2, 32
        xs = _layer(aggf, aggb, xs, cnt, wnf, wrf, lay["b_f"][None],
                    wnb, wrb, lay["b_b"][None], nchunk, width,
                    out_split=(l < 2))

    out = _pool(xs, b3, config_feat, params["W1"], params["b1"][None],
                params["W2"], params["b2"][None])
    return out[:, 0]
